# Initial kernel scaffold; baseline (speedup 1.0000x reference)
import jax
import jax.numpy as jnp
from jax.experimental import pallas as pl


def kernel(sequences, table):
    raise NotImplementedError("write your pallas kernel here")


# speedup vs baseline: 1.0373x; 1.0373x over previous
import functools

import jax
import jax.numpy as jnp
from jax import lax
from jax.experimental import pallas as pl
from jax.experimental.pallas import tpu as pltpu
from jax.experimental.pallas import tpu_sc as plsc

DIM = 16
B = 16384
L = 20
LANES = 16

NC = 2
NS = 16
NW = NC * NS

SEQ_PER_W = B // NW
IDX_PER_W = SEQ_PER_W * L
CHUNK_SEQS = 64
CHUNK_IDX = CHUNK_SEQS * L
N_CHUNKS = SEQ_PER_W // CHUNK_SEQS
G_ROWS = 128
G_PER_CHUNK = CHUNK_IDX // G_ROWS
ROWS_PER_W = IDX_PER_W // G_ROWS
IDX_PAD = 16


def _body(seq_flat_hbm, table_hbm, out_hbm,
          idx_flat, rows0, rows1, outbuf, t0_v,
          sem0, sem1):
    wid = lax.axis_index("s") * NC + lax.axis_index("c")

    pltpu.sync_copy(seq_flat_hbm.at[pl.ds(wid * IDX_PER_W, IDX_PER_W)],
                    idx_flat.at[pl.ds(0, IDX_PER_W)])
    pltpu.sync_copy(table_hbm.at[pl.ds(0, 1)], t0_v)

    rows_bufs = (rows0, rows1)
    sems = (sem0, sem1)

    def fire(chunk):
        buf = rows_bufs[chunk % 2]
        sem = sems[chunk % 2]
        cps = []
        for g in range(G_PER_CHUNK):
            idx_row = idx_flat.at[
                pl.ds((chunk * G_PER_CHUNK + g) * G_ROWS, G_ROWS)]
            dst = buf.at[pl.ds(g * G_ROWS, G_ROWS)]
            cps.append(pltpu.async_copy(table_hbm.at[idx_row], dst, sem))
        return cps

    inflight = [fire(0), fire(1)]

    lane_iota = lax.iota(jnp.int32, LANES)
    lo4 = jnp.where(lane_iota < 4, 1.0, 0.0).astype(jnp.float32)
    lo8 = jnp.where(lane_iota < 8, 1.0, 0.0).astype(jnp.float32)
    t0 = t0_v[0]
    lf = jnp.float32(L)
    one = jnp.float32(1.0)

    for chunk in range(N_CHUNKS):
        buf = rows_bufs[chunk % 2]
        for cp in inflight[chunk % 2]:
            cp.wait()

        def pair_body(p, _, chunk=chunk, buf=buf):
            rb = p * (2 * L)
            acc0 = buf[rb]
            for j in range(1, L):
                acc0 = acc0 + buf[rb + j]
            acc1 = buf[rb + L]
            for j in range(L + 1, 2 * L):
                acc1 = acc1 + buf[rb + j]

            ib = chunk * CHUNK_IDX + rb
            v0 = idx_flat[pl.ds(ib, LANES)]
            v1 = idx_flat[pl.ds(ib + LANES, LANES)]
            v2 = idx_flat[pl.ds(ib + 2 * LANES, LANES)]
            z0 = jnp.where(v0 == 0, 1.0, 0.0).astype(jnp.float32)
            z1 = jnp.where(v1 == 0, 1.0, 0.0).astype(jnp.float32)
            z2 = jnp.where(v2 == 0, 1.0, 0.0).astype(jnp.float32)
            za = jnp.full((LANES,), jnp.sum(z0) + jnp.sum(z1 * lo4))
            zb = jnp.full((LANES,),
                          jnp.sum(z1 * (one - lo4)) + jnp.sum(z2 * lo8))
            s0 = 1.0 / jnp.maximum(lf - za, 1.0)
            s1 = 1.0 / jnp.maximum(lf - zb, 1.0)

            sgw = chunk * CHUNK_SEQS + 2 * p
            outbuf[sgw] = (acc0 - za * t0) * s0
            outbuf[sgw + 1] = (acc1 - zb * t0) * s1
            return _

        lax.fori_loop(0, CHUNK_SEQS // 2, pair_body, 0, unroll=False)

        if chunk + 2 < N_CHUNKS:
            inflight[chunk % 2] = fire(chunk + 2)

    pltpu.sync_copy(outbuf, out_hbm.at[pl.ds(wid * SEQ_PER_W, SEQ_PER_W)])


@jax.jit
def _run(seq_flat, table):
    mesh = plsc.VectorSubcoreMesh(core_axis_name="c", subcore_axis_name="s")
    k = functools.partial(
        pl.kernel,
        mesh=mesh,
        out_type=jax.ShapeDtypeStruct((B, DIM), jnp.float32),
        compiler_params=pltpu.CompilerParams(
            needs_layout_passes=False, use_tc_tiling_on_sc=False),
        scratch_types=[
            pltpu.VMEM((IDX_PER_W + IDX_PAD,), jnp.int32),
            pltpu.VMEM((CHUNK_IDX, DIM), jnp.float32),
            pltpu.VMEM((CHUNK_IDX, DIM), jnp.float32),
            pltpu.VMEM((SEQ_PER_W, DIM), jnp.float32),
            pltpu.VMEM((1, DIM), jnp.float32),
            pltpu.SemaphoreType.DMA,
            pltpu.SemaphoreType.DMA,
        ],
    )(_body)
    return k(seq_flat, table)


def kernel(sequences, table):
    return _run(sequences.reshape(B * L), table)
